# Initial kernel scaffold; baseline (speedup 1.0000x reference)
import jax
import jax.numpy as jnp
from jax.experimental import pallas as pl


def kernel(x, values, row_indices, row_offsets, column_indices, bias):
    raise NotImplementedError("write your pallas kernel here")


# speedup vs baseline: 47.6784x; 47.6784x over previous
import jax
import jax.numpy as jnp
from jax.experimental import pallas as pl
from jax.experimental.pallas import tpu as pltpu

_M = 4096
_K = 4096
_NNZ_PER_ROW = 409
_TM = 512
_TK = 512
_NCOL = 256


def _mm_body(w_ref, x_ref, b_ref, o_ref, acc_ref):
    k = pl.program_id(1)

    @pl.when(k == 0)
    def _():
        acc_ref[...] = jnp.zeros_like(acc_ref)

    acc_ref[...] += jnp.dot(w_ref[...], x_ref[...],
                            preferred_element_type=jnp.float32)

    @pl.when(k == pl.num_programs(1) - 1)
    def _():
        o_ref[...] = acc_ref[...] + b_ref[0:1, :]


def _matmul(w, x2, bias_flat):
    grid = (_M // _TM, _K // _TK)
    return pl.pallas_call(
        _mm_body,
        grid=grid,
        in_specs=[
            pl.BlockSpec((_TM, _TK), lambda m, k: (m, k)),
            pl.BlockSpec((_TK, _NCOL), lambda m, k: (k, 0)),
            pl.BlockSpec((8, _NCOL), lambda m, k: (0, 0)),
        ],
        out_specs=pl.BlockSpec((_TM, _NCOL), lambda m, k: (m, 0)),
        out_shape=jax.ShapeDtypeStruct((_M, _NCOL), jnp.float32),
        scratch_shapes=[pltpu.VMEM((_TM, _NCOL), jnp.float32)],
    )(w, x2, bias_flat)


def kernel(x, values, row_indices, row_offsets, column_indices, bias):
    B, SEQ, K = x.shape
    rows = jnp.repeat(jnp.arange(_M, dtype=jnp.int32), _NNZ_PER_ROW)
    W = jnp.zeros((_M, _K), jnp.float32).at[rows, column_indices].add(values)
    x2 = jnp.transpose(x, (2, 0, 1)).reshape(K, B * SEQ)
    bias_flat = jnp.broadcast_to(jnp.tile(bias, B)[None, :], (8, B * SEQ))
    out_flat = _matmul(W, x2, bias_flat)
    return jnp.transpose(out_flat.reshape(_M, B, SEQ), (1, 0, 2))


# speedup vs baseline: 721.1590x; 15.1255x over previous
import functools

import jax
import jax.numpy as jnp
from jax import lax
from jax.experimental import pallas as pl
from jax.experimental.pallas import tpu as pltpu
from jax.experimental.pallas import tpu_sc as plsc

_M = 4096
_K = 4096
_NNZ_PER_ROW = 409
_NNZ = _M * _NNZ_PER_ROW
_NCOL = 256

_NW = 32
_ROWS_PER_W = _M // _NW
_G = 16
_NG = _ROWS_PER_W // _G
_REG = _G * _K
_NNZ_G = _G * _NNZ_PER_ROW
_CH = 128
_NCH = (_NNZ_G + _CH - 1) // _CH
_PAD_G = _NCH * _CH
_ZW = 16384

_TM = 512
_TK = 512


def _densify(vals3, idx4):
    mesh = plsc.VectorSubcoreMesh(core_axis_name="c", subcore_axis_name="s")

    @functools.partial(
        pl.kernel,
        out_type=jax.ShapeDtypeStruct((_M * _K,), jnp.float32),
        mesh=mesh,
        scratch_types=[
            pltpu.VMEM_SHARED((16 * _REG,), jnp.float32),
            pltpu.VMEM((_PAD_G,), jnp.float32),
            pltpu.VMEM((_NCH, _CH), jnp.int32),
            pltpu.VMEM((_ZW,), jnp.float32),
        ],
    )
    def k(vals_hbm, idx_hbm, w_hbm, shared, vals_v, idx_v, zero_v):
        c = lax.axis_index("c")
        s = lax.axis_index("s")
        wid = s * 2 + c
        base = s * _REG

        def zinit(i, carry):
            zero_v[pl.ds(i * 16, 16)] = jnp.zeros((16,), jnp.float32)
            return carry

        lax.fori_loop(0, _ZW // 16, zinit, 0)

        def group(g, carry):
            pltpu.sync_copy(vals_hbm.at[wid, g], vals_v)
            pltpu.sync_copy(idx_hbm.at[wid, g], idx_v)

            def zfill(z, carry2):
                pltpu.sync_copy(zero_v, shared.at[pl.ds(base + z * _ZW, _ZW)])
                return carry2

            lax.fori_loop(0, _REG // _ZW, zfill, 0)

            def scat(j, carry2):
                pltpu.sync_copy(vals_v.at[pl.ds(j * _CH, _CH)],
                                shared.at[idx_v.at[j]], add=True)
                return carry2

            lax.fori_loop(0, _NCH, scat, 0)

            row0 = (wid * _ROWS_PER_W + g * _G) * _K
            pltpu.sync_copy(shared.at[pl.ds(base, _REG)],
                            w_hbm.at[pl.ds(row0, _REG)])
            return carry

        lax.fori_loop(0, _NG, group, 0)

    return k(vals3, idx4)


def _mm_body(w_ref, x_ref, b_ref, o_ref, acc_ref):
    kk = pl.program_id(1)

    @pl.when(kk == 0)
    def _():
        acc_ref[...] = jnp.zeros_like(acc_ref)

    acc_ref[...] += jnp.dot(w_ref[...], x_ref[...],
                            preferred_element_type=jnp.float32)

    @pl.when(kk == pl.num_programs(1) - 1)
    def _():
        o_ref[...] = acc_ref[...] + b_ref[0:1, :]


def _matmul(w, x2, bias_flat):
    grid = (_M // _TM, _K // _TK)
    return pl.pallas_call(
        _mm_body,
        grid=grid,
        in_specs=[
            pl.BlockSpec((_TM, _TK), lambda m, k: (m, k)),
            pl.BlockSpec((_TK, _NCOL), lambda m, k: (k, 0)),
            pl.BlockSpec((8, _NCOL), lambda m, k: (0, 0)),
        ],
        out_specs=pl.BlockSpec((_TM, _NCOL), lambda m, k: (m, 0)),
        out_shape=jax.ShapeDtypeStruct((_M, _NCOL), jnp.float32),
        scratch_shapes=[pltpu.VMEM((_TM, _NCOL), jnp.float32)],
    )(w, x2, bias_flat)


def kernel(x, values, row_indices, row_offsets, column_indices, bias):
    B, SEQ, K = x.shape
    p = jnp.arange(_NNZ, dtype=jnp.int32)
    r = p // _NNZ_PER_ROW
    rig = r % _G
    sub = r // (2 * _ROWS_PER_W)
    idxf = sub * _REG + rig * _K + column_indices
    vals3 = jnp.pad(values.reshape(_NW, _NG, _NNZ_G),
                    ((0, 0), (0, 0), (0, _PAD_G - _NNZ_G)))
    idx4 = jnp.pad(idxf.reshape(_NW, _NG, _NNZ_G),
                   ((0, 0), (0, 0), (0, _PAD_G - _NNZ_G)))
    idx4 = idx4.reshape(_NW, _NG, _NCH, _CH)

    W = _densify(vals3, idx4).reshape(_M, _K)

    x2 = jnp.transpose(x, (2, 0, 1)).reshape(K, B * SEQ)
    bias_flat = jnp.broadcast_to(jnp.tile(bias, B)[None, :], (8, B * SEQ))
    out_flat = _matmul(W, x2, bias_flat)
    return jnp.transpose(out_flat.reshape(_M, B, SEQ), (1, 0, 2))


# speedup vs baseline: 1133.2685x; 1.5715x over previous
import functools

import jax
import jax.numpy as jnp
from jax import lax
from jax.experimental import pallas as pl
from jax.experimental.pallas import tpu as pltpu
from jax.experimental.pallas import tpu_sc as plsc

_M = 4096
_K = 4096
_NNZ_PER_ROW = 409
_NNZ = _M * _NNZ_PER_ROW
_NCOL = 256

_NW = 32
_ROWS_PER_W = _M // _NW
_G = 16
_NG = _ROWS_PER_W // _G
_REG = _G * _K
_NNZ_G = _G * _NNZ_PER_ROW
_CH = 128
_NCH = (_NNZ_G + _CH - 1) // _CH
_PAD_G = _NCH * _CH
_ZW = 16384

_TM = 512
_TK = 512


def _densify(vals3, idx4):
    mesh = plsc.VectorSubcoreMesh(core_axis_name="c", subcore_axis_name="s")

    @functools.partial(
        pl.kernel,
        out_type=jax.ShapeDtypeStruct((_M, _K), jnp.float32),
        mesh=mesh,
        scratch_types=[
            pltpu.VMEM_SHARED((16 * _REG + 8,), jnp.float32),
            pltpu.VMEM((_PAD_G,), jnp.float32),
            pltpu.VMEM((_NCH, _CH), jnp.int32),
            pltpu.VMEM((_ZW,), jnp.float32),
            pltpu.SemaphoreType.DMA,
            pltpu.SemaphoreType.DMA,
            pltpu.SemaphoreType.DMA,
        ],
    )
    def k(vals_hbm, idx_hbm, w_hbm, shared, vals_v, idx_v, zero_v,
          zsem, ssem, osem):
        c = lax.axis_index("c")
        s = lax.axis_index("s")
        wid = s * 2 + c
        base = s * _REG
        row_base = wid * _ROWS_PER_W

        def zinit(i, carry):
            zero_v[pl.ds(i * 16, 16)] = jnp.zeros((16,), jnp.float32)
            return carry

        lax.fori_loop(0, _ZW // 16, zinit, 0)

        def drain_out(g):
            def d(i, carry):
                pltpu.make_async_copy(shared.at[pl.ds(base + i * _K, _K)],
                                      w_hbm.at[row_base + g * _G + i],
                                      osem).wait()
                return carry
            lax.fori_loop(0, _G, d, 0)

        def group(g, carry):
            @pl.when(g > 0)
            def _():
                drain_out(g - 1)

            def zf(z, carry2):
                pltpu.async_copy(zero_v, shared.at[pl.ds(base + z * _ZW, _ZW)],
                                 zsem)
                return carry2

            lax.fori_loop(0, _REG // _ZW, zf, 0)

            pltpu.sync_copy(vals_hbm.at[wid, g], vals_v)
            pltpu.sync_copy(idx_hbm.at[wid, g], idx_v)

            def zd(z, carry2):
                pltpu.make_async_copy(zero_v,
                                      shared.at[pl.ds(base + z * _ZW, _ZW)],
                                      zsem).wait()
                return carry2

            lax.fori_loop(0, _REG // _ZW, zd, 0)

            def sc(j, carry2):
                pltpu.async_copy(vals_v.at[pl.ds(j * _CH, _CH)],
                                 shared.at[idx_v.at[j]], ssem, add=True)
                return carry2

            def sd(j, carry2):
                pltpu.make_async_copy(vals_v.at[pl.ds(j * _CH, _CH)],
                                      shared.at[idx_v.at[j]], ssem).wait()
                return carry2

            lax.fori_loop(0, _NCH // 2, lambda j, c2: sc(2 * j, c2), 0)
            lax.fori_loop(0, _NCH // 2, lambda j, c2: sd(2 * j, c2), 0)
            lax.fori_loop(0, _NCH // 2, lambda j, c2: sc(2 * j + 1, c2), 0)
            lax.fori_loop(0, _NCH // 2, lambda j, c2: sd(2 * j + 1, c2), 0)

            def of(i, carry2):
                pltpu.async_copy(shared.at[pl.ds(base + i * _K, _K)],
                                 w_hbm.at[row_base + g * _G + i], osem)
                return carry2

            lax.fori_loop(0, _G, of, 0)
            return carry

        lax.fori_loop(0, _NG, group, 0)
        drain_out(_NG - 1)

    return k(vals3, idx4)


def _mm_body(w_ref, x_ref, b_ref, o_ref, acc_ref):
    kk = pl.program_id(1)

    @pl.when(kk == 0)
    def _():
        acc_ref[...] = jnp.zeros_like(acc_ref)

    acc_ref[...] += jnp.dot(w_ref[...], x_ref[...],
                            preferred_element_type=jnp.float32)

    @pl.when(kk == pl.num_programs(1) - 1)
    def _():
        o_ref[...] = acc_ref[...] + b_ref[0:1, :]


def _matmul(w, x2, bias_flat):
    grid = (_M // _TM, _K // _TK)
    return pl.pallas_call(
        _mm_body,
        grid=grid,
        in_specs=[
            pl.BlockSpec((_TM, _TK), lambda m, k: (m, k)),
            pl.BlockSpec((_TK, _NCOL), lambda m, k: (k, 0)),
            pl.BlockSpec((8, _NCOL), lambda m, k: (0, 0)),
        ],
        out_specs=pl.BlockSpec((_TM, _NCOL), lambda m, k: (m, 0)),
        out_shape=jax.ShapeDtypeStruct((_M, _NCOL), jnp.float32),
        scratch_shapes=[pltpu.VMEM((_TM, _NCOL), jnp.float32)],
    )(w, x2, bias_flat)


def kernel(x, values, row_indices, row_offsets, column_indices, bias):
    B, SEQ, K = x.shape
    p = jnp.arange(_NNZ, dtype=jnp.int32)
    r = p // _NNZ_PER_ROW
    rig = r % _G
    sub = r // (2 * _ROWS_PER_W)
    idxf = sub * _REG + rig * _K + column_indices
    vals3 = jnp.pad(values.reshape(_NW, _NG, _NNZ_G),
                    ((0, 0), (0, 0), (0, _PAD_G - _NNZ_G)))
    idx4 = jnp.pad(idxf.reshape(_NW, _NG, _NNZ_G),
                   ((0, 0), (0, 0), (0, _PAD_G - _NNZ_G)),
                   constant_values=16 * _REG)
    idx4 = idx4.reshape(_NW, _NG, _NCH, _CH)

    W = _densify(vals3, idx4)

    x2 = jnp.transpose(x, (2, 0, 1)).reshape(K, B * SEQ)
    bias_flat = jnp.broadcast_to(jnp.tile(bias, B)[None, :], (8, B * SEQ))
    out_flat = _matmul(W, x2, bias_flat)
    return jnp.transpose(out_flat.reshape(_M, B, SEQ), (1, 0, 2))


# speedup vs baseline: 1306.5504x; 1.1529x over previous
import functools

import jax
import jax.numpy as jnp
import numpy as np
from jax import lax
from jax.experimental import pallas as pl
from jax.experimental.pallas import tpu as pltpu
from jax.experimental.pallas import tpu_sc as plsc

_M = 4096
_K = 4096
_NNZ_PER_ROW = 409
_NNZ = _M * _NNZ_PER_ROW
_NCOL = 256

_NW = 32
_ROWS_PER_W = _M // _NW
_G = 16
_NG = _ROWS_PER_W // _G
_REG = _G * _K
_NNZ_G = _G * _NNZ_PER_ROW
_CH = 128
_NCH = (_NNZ_G + _CH - 1) // _CH
_PAD_G = _NCH * _CH
_ZW = 16384

_ROW_OF_NNZ = np.arange(_NNZ, dtype=np.int64) // _NNZ_PER_ROW
_BASE_FLAT = ((_ROW_OF_NNZ % _G) * _K
              + (_ROW_OF_NNZ // (2 * _ROWS_PER_W)) * _REG).astype(np.int32)
_BASE_PAD = np.full((_NW, _NG, _PAD_G), 16 * _REG, dtype=np.int32)
_BASE_PAD[:, :, :_NNZ_G] = _BASE_FLAT.reshape(_NW, _NG, _NNZ_G)

_TM = 512
_TK = 512


def _densify(vals_flat, idx3):
    mesh = plsc.VectorSubcoreMesh(core_axis_name="c", subcore_axis_name="s")

    @functools.partial(
        pl.kernel,
        out_type=jax.ShapeDtypeStruct((_M, _K), jnp.float32),
        mesh=mesh,
        scratch_types=[
            pltpu.VMEM_SHARED((16 * _REG + 8,), jnp.float32),
            pltpu.VMEM((_PAD_G,), jnp.int32),
            pltpu.VMEM((_PAD_G,), jnp.float32),
            pltpu.VMEM((_ZW,), jnp.float32),
            pltpu.SemaphoreType.DMA,
            pltpu.SemaphoreType.DMA,
            pltpu.SemaphoreType.DMA,
        ],
    )
    def k(vals_hbm, idx_hbm, w_hbm, shared, idx_v, vals_v, zero_v,
          zsem, ssem, osem):
        c = lax.axis_index("c")
        s = lax.axis_index("s")
        wid = s * 2 + c
        base = s * _REG
        row_base = wid * _ROWS_PER_W
        nnz_base = wid * (_NG * _NNZ_G)

        def zinit(i, carry):
            zero_v[pl.ds(i * 16, 16)] = jnp.zeros((16,), jnp.float32)
            return carry

        lax.fori_loop(0, _ZW // 16, zinit, 0)

        def drain_out(g):
            def d(i, carry):
                pltpu.make_async_copy(shared.at[pl.ds(base + i * _K, _K)],
                                      w_hbm.at[row_base + g * _G + i],
                                      osem).wait()
                return carry
            lax.fori_loop(0, _G, d, 0)

        def group(g, carry):
            @pl.when(g > 0)
            def _():
                drain_out(g - 1)

            def zf(z, carry2):
                pltpu.async_copy(zero_v, shared.at[pl.ds(base + z * _ZW, _ZW)],
                                 zsem)
                return carry2

            lax.fori_loop(0, _REG // _ZW, zf, 0)
            pltpu.sync_copy(idx_hbm.at[wid, g], idx_v)
            pltpu.sync_copy(vals_hbm.at[pl.ds(nnz_base + g * _NNZ_G, _PAD_G)],
                            vals_v)

            def zd(z, carry2):
                pltpu.make_async_copy(zero_v,
                                      shared.at[pl.ds(base + z * _ZW, _ZW)],
                                      zsem).wait()
                return carry2

            lax.fori_loop(0, _REG // _ZW, zd, 0)

            pltpu.async_copy(vals_v, shared.at[idx_v], ssem, add=True)
            pltpu.make_async_copy(vals_v, shared.at[idx_v], ssem).wait()

            def of(i, carry2):
                pltpu.async_copy(shared.at[pl.ds(base + i * _K, _K)],
                                 w_hbm.at[row_base + g * _G + i], osem)
                return carry2

            lax.fori_loop(0, _G, of, 0)
            return carry

        lax.fori_loop(0, _NG, group, 0)
        drain_out(_NG - 1)

    return k(vals_flat, idx3)


def _mm_body(w_ref, x_ref, b_ref, o_ref, acc_ref):
    kk = pl.program_id(1)

    @pl.when(kk == 0)
    def _():
        acc_ref[...] = jnp.zeros_like(acc_ref)

    acc_ref[...] += jnp.dot(w_ref[...], x_ref[...],
                            preferred_element_type=jnp.float32)

    @pl.when(kk == pl.num_programs(1) - 1)
    def _():
        o_ref[...] = acc_ref[...] + b_ref[0:1, :]


def _matmul(w, x2, bias_flat):
    grid = (_M // _TM, _K // _TK)
    return pl.pallas_call(
        _mm_body,
        grid=grid,
        in_specs=[
            pl.BlockSpec((_TM, _TK), lambda m, k: (m, k)),
            pl.BlockSpec((_TK, _NCOL), lambda m, k: (k, 0)),
            pl.BlockSpec((8, _NCOL), lambda m, k: (0, 0)),
        ],
        out_specs=pl.BlockSpec((_TM, _NCOL), lambda m, k: (m, 0)),
        out_shape=jax.ShapeDtypeStruct((_M, _NCOL), jnp.float32),
        scratch_shapes=[pltpu.VMEM((_TM, _NCOL), jnp.float32)],
    )(w, x2, bias_flat)


def kernel(x, values, row_indices, row_offsets, column_indices, bias):
    B, SEQ, K = x.shape
    cols_pad = jnp.pad(column_indices.reshape(_NW, _NG, _NNZ_G),
                       ((0, 0), (0, 0), (0, _PAD_G - _NNZ_G)))
    idx3 = jnp.asarray(_BASE_PAD) + cols_pad
    vals_flat = jnp.pad(values, (0, _CH))

    W = _densify(vals_flat, idx3)

    x2 = jnp.transpose(x, (2, 0, 1)).reshape(K, B * SEQ)
    bias_flat = jnp.broadcast_to(jnp.tile(bias, B)[None, :], (8, B * SEQ))
    out_flat = _matmul(W, x2, bias_flat)
    return jnp.transpose(out_flat.reshape(_M, B, SEQ), (1, 0, 2))


# speedup vs baseline: 1646.4807x; 1.2602x over previous
import functools

import jax
import jax.numpy as jnp
import numpy as np
from jax import lax
from jax.experimental import pallas as pl
from jax.experimental.pallas import tpu as pltpu
from jax.experimental.pallas import tpu_sc as plsc

_M = 4096
_K = 4096
_NNZ_PER_ROW = 409
_NNZ = _M * _NNZ_PER_ROW
_NCOL = 256

_NW = 32
_ROWS_PER_W = _M // _NW
_G = 16
_NG = _ROWS_PER_W // _G
_REG = _G * _K
_NNZ_G = _G * _NNZ_PER_ROW
_CH = 128
_NCH = (_NNZ_G + _CH - 1) // _CH
_PAD_G = _NCH * _CH
_ZW = 16384

_ROW_OF_NNZ = np.arange(_NNZ, dtype=np.int64) // _NNZ_PER_ROW
_BASE_FLAT = ((_ROW_OF_NNZ % _G) * _K
              + (_ROW_OF_NNZ // (2 * _ROWS_PER_W)) * _REG).astype(np.int32)
_BASE_PAD = np.full((_NW, _NG, _PAD_G), 16 * _REG, dtype=np.int32)
_BASE_PAD[:, :, :_NNZ_G] = _BASE_FLAT.reshape(_NW, _NG, _NNZ_G)

_TM = 256


def _densify(vals_flat, idx3):
    mesh = plsc.VectorSubcoreMesh(core_axis_name="c", subcore_axis_name="s")

    @functools.partial(
        pl.kernel,
        out_type=jax.ShapeDtypeStruct((_M, _K), jnp.float32),
        mesh=mesh,
        scratch_types=[
            pltpu.VMEM_SHARED((16 * _REG + 8,), jnp.float32),
            pltpu.VMEM((_PAD_G,), jnp.int32),
            pltpu.VMEM((_PAD_G,), jnp.float32),
            pltpu.VMEM((_ZW,), jnp.float32),
            pltpu.SemaphoreType.DMA,
            pltpu.SemaphoreType.DMA,
            pltpu.SemaphoreType.DMA,
        ],
    )
    def k(vals_hbm, idx_hbm, w_hbm, shared, idx_v, vals_v, zero_v,
          zsem, ssem, osem):
        c = lax.axis_index("c")
        s = lax.axis_index("s")
        wid = s * 2 + c
        base = s * _REG
        row_base = wid * _ROWS_PER_W
        nnz_base = wid * (_NG * _NNZ_G)

        def zinit(i, carry):
            zero_v[pl.ds(i * 16, 16)] = jnp.zeros((16,), jnp.float32)
            return carry

        lax.fori_loop(0, _ZW // 16, zinit, 0)

        def drain_out(g):
            def d(i, carry):
                pltpu.make_async_copy(shared.at[pl.ds(base + i * _K, _K)],
                                      w_hbm.at[row_base + g * _G + i],
                                      osem).wait()
                return carry
            lax.fori_loop(0, _G, d, 0)

        def group(g, carry):
            @pl.when(g > 0)
            def _():
                drain_out(g - 1)

            def zf(z, carry2):
                pltpu.async_copy(zero_v, shared.at[pl.ds(base + z * _ZW, _ZW)],
                                 zsem)
                return carry2

            lax.fori_loop(0, _REG // _ZW, zf, 0)
            pltpu.sync_copy(idx_hbm.at[wid, g], idx_v)
            pltpu.sync_copy(vals_hbm.at[pl.ds(nnz_base + g * _NNZ_G, _PAD_G)],
                            vals_v)

            def zd(z, carry2):
                pltpu.make_async_copy(zero_v,
                                      shared.at[pl.ds(base + z * _ZW, _ZW)],
                                      zsem).wait()
                return carry2

            lax.fori_loop(0, _REG // _ZW, zd, 0)

            pltpu.async_copy(vals_v, shared.at[idx_v], ssem, add=True)
            pltpu.make_async_copy(vals_v, shared.at[idx_v], ssem).wait()

            def of(i, carry2):
                pltpu.async_copy(shared.at[pl.ds(base + i * _K, _K)],
                                 w_hbm.at[row_base + g * _G + i], osem)
                return carry2

            lax.fori_loop(0, _G, of, 0)
            return carry

        lax.fori_loop(0, _NG, group, 0)
        drain_out(_NG - 1)

    return k(vals_flat, idx3)


def _mm_body(w_ref, x_ref, b_ref, o_ref):
    o_ref[...] = jnp.dot(w_ref[...], x_ref[...],
                         preferred_element_type=jnp.float32) + b_ref[0:1, :]


def _matmul(w, x2, bias_flat):
    grid = (_M // _TM,)
    return pl.pallas_call(
        _mm_body,
        grid=grid,
        in_specs=[
            pl.BlockSpec((_TM, _K), lambda m: (m, 0)),
            pl.BlockSpec((_K, _NCOL), lambda m: (0, 0)),
            pl.BlockSpec((8, _NCOL), lambda m: (0, 0)),
        ],
        out_specs=pl.BlockSpec((_TM, _NCOL), lambda m: (m, 0)),
        out_shape=jax.ShapeDtypeStruct((_M, _NCOL), jnp.float32),
    )(w, x2, bias_flat)


def kernel(x, values, row_indices, row_offsets, column_indices, bias):
    B, SEQ, K = x.shape
    cols_pad = jnp.pad(column_indices.reshape(_NW, _NG, _NNZ_G),
                       ((0, 0), (0, 0), (0, _PAD_G - _NNZ_G)))
    idx3 = jnp.asarray(_BASE_PAD) + cols_pad
    vals_flat = jnp.pad(values, (0, _CH))

    W = _densify(vals_flat, idx3)

    x2 = jnp.transpose(x, (2, 0, 1)).reshape(K, B * SEQ)
    bias_flat = jnp.broadcast_to(jnp.tile(bias, B)[None, :], (8, B * SEQ))
    out_flat = _matmul(W, x2, bias_flat)
    return jnp.transpose(out_flat.reshape(_M, B, SEQ), (1, 0, 2))


# speedup vs baseline: 1649.1319x; 1.0016x over previous
import functools

import jax
import jax.numpy as jnp
import numpy as np
from jax import lax
from jax.experimental import pallas as pl
from jax.experimental.pallas import tpu as pltpu
from jax.experimental.pallas import tpu_sc as plsc

_M = 4096
_K = 4096
_NNZ_PER_ROW = 409
_NNZ = _M * _NNZ_PER_ROW
_NCOL = 256

_NW = 32
_ROWS_PER_W = _M // _NW
_G = 16
_NG = _ROWS_PER_W // _G
_REG = _G * _K
_NNZ_G = _G * _NNZ_PER_ROW
_CH = 128
_NCH = (_NNZ_G + _CH - 1) // _CH
_PAD_G = _NCH * _CH
_ZW = 16384

_ROW_OF_NNZ = np.arange(_NNZ, dtype=np.int64) // _NNZ_PER_ROW
_BASE_FLAT = ((_ROW_OF_NNZ % _G) * _K
              + (_ROW_OF_NNZ // (2 * _ROWS_PER_W)) * _REG).astype(np.int32)
_BASE_PAD = np.full((_NW, _NG, _PAD_G), 16 * _REG, dtype=np.int32)
_BASE_PAD[:, :, :_NNZ_G] = _BASE_FLAT.reshape(_NW, _NG, _NNZ_G)

_TM = 256


def _densify(vals_flat, idx3):
    mesh = plsc.VectorSubcoreMesh(core_axis_name="c", subcore_axis_name="s")

    @functools.partial(
        pl.kernel,
        out_type=jax.ShapeDtypeStruct((_M, _K), jnp.float32),
        mesh=mesh,
        scratch_types=[
            pltpu.VMEM_SHARED((16 * _REG + 8,), jnp.float32),
            pltpu.VMEM((_PAD_G,), jnp.int32),
            pltpu.VMEM((_PAD_G,), jnp.float32),
            pltpu.VMEM((_ZW,), jnp.float32),
            pltpu.SemaphoreType.DMA,
            pltpu.SemaphoreType.DMA,
            pltpu.SemaphoreType.DMA,
        ],
    )
    def k(zeros_hbm, vals_hbm, idx_hbm, w_hbm, shared, idx_v, vals_v, zero_v,
          zsem, ssem, osem):
        c = lax.axis_index("c")
        s = lax.axis_index("s")
        wid = s * 2 + c
        base = s * _REG
        row_base = wid * _ROWS_PER_W
        nnz_base = wid * (_NG * _NNZ_G)

        pltpu.sync_copy(zeros_hbm, zero_v)

        def drain_out(g):
            def d(i, carry):
                pltpu.make_async_copy(shared.at[pl.ds(base + i * _K, _K)],
                                      w_hbm.at[row_base + g * _G + i],
                                      osem).wait()
                return carry
            lax.fori_loop(0, _G, d, 0)

        def group(g, carry):
            @pl.when(g > 0)
            def _():
                drain_out(g - 1)

            def zf(z, carry2):
                pltpu.async_copy(zero_v, shared.at[pl.ds(base + z * _ZW, _ZW)],
                                 zsem)
                return carry2

            lax.fori_loop(0, _REG // _ZW, zf, 0)
            pltpu.sync_copy(idx_hbm.at[wid, g], idx_v)
            pltpu.sync_copy(vals_hbm.at[pl.ds(nnz_base + g * _NNZ_G, _PAD_G)],
                            vals_v)

            def zd(z, carry2):
                pltpu.make_async_copy(zero_v,
                                      shared.at[pl.ds(base + z * _ZW, _ZW)],
                                      zsem).wait()
                return carry2

            lax.fori_loop(0, _REG // _ZW, zd, 0)

            pltpu.async_copy(vals_v, shared.at[idx_v], ssem, add=True)
            pltpu.make_async_copy(vals_v, shared.at[idx_v], ssem).wait()

            def of(i, carry2):
                pltpu.async_copy(shared.at[pl.ds(base + i * _K, _K)],
                                 w_hbm.at[row_base + g * _G + i], osem)
                return carry2

            lax.fori_loop(0, _G, of, 0)
            return carry

        lax.fori_loop(0, _NG, group, 0)
        drain_out(_NG - 1)

    return k(jnp.zeros((_ZW,), jnp.float32), vals_flat, idx3)


def _mm_body(w_ref, x_ref, b_ref, o_ref):
    o_ref[...] = jnp.dot(w_ref[...], x_ref[...],
                         preferred_element_type=jnp.float32) + b_ref[0:1, :]


def _matmul(w, x2, bias_flat):
    grid = (_M // _TM,)
    return pl.pallas_call(
        _mm_body,
        grid=grid,
        in_specs=[
            pl.BlockSpec((_TM, _K), lambda m: (m, 0)),
            pl.BlockSpec((_K, _NCOL), lambda m: (0, 0)),
            pl.BlockSpec((8, _NCOL), lambda m: (0, 0)),
        ],
        out_specs=pl.BlockSpec((_TM, _NCOL), lambda m: (m, 0)),
        out_shape=jax.ShapeDtypeStruct((_M, _NCOL), jnp.float32),
    )(w, x2, bias_flat)


def kernel(x, values, row_indices, row_offsets, column_indices, bias):
    B, SEQ, K = x.shape
    cols_pad = jnp.pad(column_indices.reshape(_NW, _NG, _NNZ_G),
                       ((0, 0), (0, 0), (0, _PAD_G - _NNZ_G)))
    idx3 = jnp.asarray(_BASE_PAD) + cols_pad
    vals_flat = jnp.pad(values, (0, _CH))

    W = _densify(vals_flat, idx3)

    x2 = jnp.transpose(x, (2, 0, 1)).reshape(K, B * SEQ)
    bias_flat = jnp.broadcast_to(jnp.tile(bias, B)[None, :], (8, B * SEQ))
    out_flat = _matmul(W, x2, bias_flat)
    return jnp.transpose(out_flat.reshape(_M, B, SEQ), (1, 0, 2))
